# baseline (speedup 1.0000x reference)
import functools

import jax
import jax.numpy as jnp
from jax import lax
from jax.experimental import pallas as pl
from jax.experimental.pallas import tpu as pltpu
from jax.experimental.pallas import tpu_sc as plsc

D = 32
V = 1000000
B = 4096
GT = 16
GRID = B // GT
PW = GRID * 128
NC = 2
NS = 16
L = 16
NW = NC * NS
BPW = B // NW
CPW = PW // NW


def _tc_body(idxw_s, idxc_s, *refs):
    w_refs = refs[:GT]
    c_refs = refs[GT:2 * GT]
    out_ref = refs[2 * GT]
    i = pl.program_id(0)
    rowi = lax.broadcasted_iota(jnp.int32, (128, 1), 0)
    prods = []
    for k in range(GT):
        s = i * GT + k
        ohw = jnp.where(rowi == (idxw_s[s] & 127), 1.0, 0.0)
        ohc = jnp.where(rowi == (idxc_s[s] & 127), 1.0, 0.0)
        wsel = jax.lax.dot_general(
            w_refs[k][...], ohw, (((1,), (0,)), ((), ())),
            preferred_element_type=jnp.float32,
            precision=jax.lax.Precision.HIGHEST)
        csel = jax.lax.dot_general(
            c_refs[k][...], ohc, (((1,), (0,)), ((), ())),
            preferred_element_type=jnp.float32,
            precision=jax.lax.Precision.HIGHEST)
        prods.append(wsel * csel)
    out_ref[:, pl.ds(0, GT)] = jnp.concatenate(prods, axis=1)


def _w_map(k):
    return lambda i, sw, sc: (0, sw[i * GT + k] >> 7)


def _c_map(k):
    return lambda i, sw, sc: (0, sc[i * GT + k] >> 7)


_tc_gather = pl.pallas_call(
    _tc_body,
    grid_spec=pltpu.PrefetchScalarGridSpec(
        num_scalar_prefetch=2,
        grid=(GRID,),
        in_specs=(
            [pl.BlockSpec((D, 128), _w_map(k)) for k in range(GT)]
            + [pl.BlockSpec((D, 128), _c_map(k)) for k in range(GT)]
        ),
        out_specs=pl.BlockSpec((D, 128), lambda i, sw, sc: (0, i)),
    ),
    out_shape=jax.ShapeDtypeStruct((D, PW), jnp.float32),
    compiler_params=pltpu.CompilerParams(
        dimension_semantics=("arbitrary",)),
)

_mesh = plsc.VectorSubcoreMesh(core_axis_name="c", subcore_axis_name="s")


@functools.partial(
    pl.kernel,
    mesh=_mesh,
    out_type=jax.ShapeDtypeStruct((B,), jnp.float32),
    scratch_types=[
        pltpu.VMEM((D, CPW), jnp.float32),
        pltpu.VMEM((BPW,), jnp.float32),
    ],
    compiler_params=pltpu.CompilerParams(needs_layout_passes=False),
)
def _sc_reduce(prod_hbm, out_hbm, slab_v, out_v):
    wid = lax.axis_index("s") * NC + lax.axis_index("c")
    pltpu.sync_copy(prod_hbm.at[:, pl.ds(wid * CPW, CPW)], slab_v)
    for g in range(BPW // L):
        acc = jnp.zeros((L,), jnp.float32)
        for d in range(D):
            acc = acc + slab_v[d, pl.ds(g * 128, L)]
        out_v[pl.ds(g * L, L)] = acc
    pltpu.sync_copy(out_v, out_hbm.at[pl.ds(wid * BPW, BPW)])


def kernel(inputs, word_embeddings, context_embeddings):
    idx_word = inputs[:, 1].astype(jnp.int32)
    idx_ctx = inputs[:, 0].astype(jnp.int32)
    prods = _tc_gather(idx_word, idx_ctx,
                       *([word_embeddings.T] * GT),
                       *([context_embeddings.T] * GT))
    return _sc_reduce(prods)


# speedup vs baseline: 2.4857x; 2.4857x over previous
import functools

import jax
import jax.numpy as jnp
from jax import lax
from jax.experimental import pallas as pl
from jax.experimental.pallas import tpu as pltpu
from jax.experimental.pallas import tpu_sc as plsc

D = 32
V = 1000000
B = 4096
GT = 16
GRID = B // GT
PW = GRID * 128
NC = 2
NS = 16
L = 16
NW = NC * NS
BPW = B // NW
CPW = PW // NW


def _tc_body(idxw_s, idxc_s, *refs):
    w_refs = refs[:GT]
    c_refs = refs[GT:2 * GT]
    out_ref = refs[2 * GT]
    i = pl.program_id(0)
    prods = []
    for k in range(GT):
        s = i * GT + k
        wsel = pltpu.roll(w_refs[k][...], -(idxw_s[s] & 127), 1)[:, :1]
        csel = pltpu.roll(c_refs[k][...], -(idxc_s[s] & 127), 1)[:, :1]
        prods.append(wsel * csel)
    out_ref[:, pl.ds(0, GT)] = jnp.concatenate(prods, axis=1)


def _w_map(k):
    return lambda i, sw, sc: (0, sw[i * GT + k] >> 7)


def _c_map(k):
    return lambda i, sw, sc: (0, sc[i * GT + k] >> 7)


_tc_gather = pl.pallas_call(
    _tc_body,
    grid_spec=pltpu.PrefetchScalarGridSpec(
        num_scalar_prefetch=2,
        grid=(GRID,),
        in_specs=(
            [pl.BlockSpec((D, 128), _w_map(k)) for k in range(GT)]
            + [pl.BlockSpec((D, 128), _c_map(k)) for k in range(GT)]
        ),
        out_specs=pl.BlockSpec((D, 128), lambda i, sw, sc: (0, i)),
    ),
    out_shape=jax.ShapeDtypeStruct((D, PW), jnp.float32),
    compiler_params=pltpu.CompilerParams(
        dimension_semantics=("arbitrary",)),
)

_mesh = plsc.VectorSubcoreMesh(core_axis_name="c", subcore_axis_name="s")


@functools.partial(
    pl.kernel,
    mesh=_mesh,
    out_type=jax.ShapeDtypeStruct((B,), jnp.float32),
    scratch_types=[
        pltpu.VMEM((D, CPW), jnp.float32),
        pltpu.VMEM((BPW,), jnp.float32),
    ],
    compiler_params=pltpu.CompilerParams(needs_layout_passes=False),
)
def _sc_reduce(prod_hbm, out_hbm, slab_v, out_v):
    wid = lax.axis_index("s") * NC + lax.axis_index("c")
    pltpu.sync_copy(prod_hbm.at[:, pl.ds(wid * CPW, CPW)], slab_v)
    for g in range(BPW // L):
        acc = jnp.zeros((L,), jnp.float32)
        for d in range(D):
            acc = acc + slab_v[d, pl.ds(g * 128, L)]
        out_v[pl.ds(g * L, L)] = acc
    pltpu.sync_copy(out_v, out_hbm.at[pl.ds(wid * BPW, BPW)])


def kernel(inputs, word_embeddings, context_embeddings):
    idx_word = inputs[:, 1].astype(jnp.int32)
    idx_ctx = inputs[:, 0].astype(jnp.int32)
    prods = _tc_gather(idx_word, idx_ctx,
                       *([word_embeddings.T] * GT),
                       *([context_embeddings.T] * GT))
    return _sc_reduce(prods)


# speedup vs baseline: 2.7332x; 1.0996x over previous
import functools

import jax
import jax.numpy as jnp
from jax import lax
from jax.experimental import pallas as pl
from jax.experimental.pallas import tpu as pltpu
from jax.experimental.pallas import tpu_sc as plsc

D = 32
V = 1000000
B = 4096
GT = 32
GRID = B // GT
PW = GRID * 128
NC = 2
NS = 16
L = 16
NW = NC * NS
BPW = B // NW
CPW = PW // NW


def _tc_body(idxw_s, idxc_s, *refs):
    w_refs = refs[:GT]
    c_refs = refs[GT:2 * GT]
    out_ref = refs[2 * GT]
    i = pl.program_id(0)
    lane = lax.broadcasted_iota(jnp.int32, (D, 128), 1)
    prods = []
    for k in range(GT):
        s = i * GT + k
        wsel = jnp.sum(jnp.where(lane == (idxw_s[s] & 127),
                                 w_refs[k][...], 0.0),
                       axis=1, keepdims=True)
        csel = jnp.sum(jnp.where(lane == (idxc_s[s] & 127),
                                 c_refs[k][...], 0.0),
                       axis=1, keepdims=True)
        prods.append(wsel * csel)
    out_ref[:, pl.ds(0, GT)] = jnp.concatenate(prods, axis=1)


def _w_map(k):
    return lambda i, sw, sc: (0, sw[i * GT + k] >> 7)


def _c_map(k):
    return lambda i, sw, sc: (0, sc[i * GT + k] >> 7)


_tc_gather = pl.pallas_call(
    _tc_body,
    grid_spec=pltpu.PrefetchScalarGridSpec(
        num_scalar_prefetch=2,
        grid=(GRID,),
        in_specs=(
            [pl.BlockSpec((D, 128), _w_map(k)) for k in range(GT)]
            + [pl.BlockSpec((D, 128), _c_map(k)) for k in range(GT)]
        ),
        out_specs=pl.BlockSpec((D, 128), lambda i, sw, sc: (0, i)),
    ),
    out_shape=jax.ShapeDtypeStruct((D, PW), jnp.float32),
    compiler_params=pltpu.CompilerParams(
        dimension_semantics=("arbitrary",)),
)

_mesh = plsc.VectorSubcoreMesh(core_axis_name="c", subcore_axis_name="s")


@functools.partial(
    pl.kernel,
    mesh=_mesh,
    out_type=jax.ShapeDtypeStruct((B,), jnp.float32),
    scratch_types=[
        pltpu.VMEM((D, CPW), jnp.float32),
        pltpu.VMEM((BPW,), jnp.float32),
    ],
    compiler_params=pltpu.CompilerParams(needs_layout_passes=False),
)
def _sc_reduce(prod_hbm, out_hbm, slab_v, out_v):
    wid = lax.axis_index("s") * NC + lax.axis_index("c")
    pltpu.sync_copy(prod_hbm.at[:, pl.ds(wid * CPW, CPW)], slab_v)
    for g in range(BPW // L):
        col = (g * L // GT) * 128 + (g * L) % GT
        acc = jnp.zeros((L,), jnp.float32)
        for d in range(D):
            acc = acc + slab_v[d, pl.ds(col, L)]
        out_v[pl.ds(g * L, L)] = acc
    pltpu.sync_copy(out_v, out_hbm.at[pl.ds(wid * BPW, BPW)])


def kernel(inputs, word_embeddings, context_embeddings):
    idx_word = inputs[:, 1].astype(jnp.int32)
    idx_ctx = inputs[:, 0].astype(jnp.int32)
    prods = _tc_gather(idx_word, idx_ctx,
                       *([word_embeddings.T] * GT),
                       *([context_embeddings.T] * GT))
    return _sc_reduce(prods)


# speedup vs baseline: 2.7798x; 1.0170x over previous
import functools

import jax
import jax.numpy as jnp
from jax import lax
from jax.experimental import pallas as pl
from jax.experimental.pallas import tpu as pltpu
from jax.experimental.pallas import tpu_sc as plsc

D = 32
V = 1000000
B = 4096
GT = 64
GRID = B // GT
PW = GRID * 128
NC = 2
NS = 16
L = 16
NW = NC * NS
BPW = B // NW
CPW = PW // NW


def _tc_body(idxw_s, idxc_s, *refs):
    w_refs = refs[:GT]
    c_refs = refs[GT:2 * GT]
    out_ref = refs[2 * GT]
    i = pl.program_id(0)
    lane = lax.broadcasted_iota(jnp.int32, (D, 128), 1)
    prods = []
    for k in range(GT):
        s = i * GT + k
        wsel = jnp.sum(jnp.where(lane == (idxw_s[s] & 127),
                                 w_refs[k][...], 0.0),
                       axis=1, keepdims=True)
        csel = jnp.sum(jnp.where(lane == (idxc_s[s] & 127),
                                 c_refs[k][...], 0.0),
                       axis=1, keepdims=True)
        prods.append(wsel * csel)
    out_ref[:, pl.ds(0, GT)] = jnp.concatenate(prods, axis=1)


def _w_map(k):
    return lambda i, sw, sc: (0, sw[i * GT + k] >> 7)


def _c_map(k):
    return lambda i, sw, sc: (0, sc[i * GT + k] >> 7)


_tc_gather = pl.pallas_call(
    _tc_body,
    grid_spec=pltpu.PrefetchScalarGridSpec(
        num_scalar_prefetch=2,
        grid=(GRID,),
        in_specs=(
            [pl.BlockSpec((D, 128), _w_map(k)) for k in range(GT)]
            + [pl.BlockSpec((D, 128), _c_map(k)) for k in range(GT)]
        ),
        out_specs=pl.BlockSpec((D, 128), lambda i, sw, sc: (0, i)),
    ),
    out_shape=jax.ShapeDtypeStruct((D, PW), jnp.float32),
    compiler_params=pltpu.CompilerParams(
        dimension_semantics=("arbitrary",)),
)

_mesh = plsc.VectorSubcoreMesh(core_axis_name="c", subcore_axis_name="s")


@functools.partial(
    pl.kernel,
    mesh=_mesh,
    out_type=jax.ShapeDtypeStruct((B,), jnp.float32),
    scratch_types=[
        pltpu.VMEM((D, CPW), jnp.float32),
        pltpu.VMEM((BPW,), jnp.float32),
    ],
    compiler_params=pltpu.CompilerParams(needs_layout_passes=False),
)
def _sc_reduce(prod_hbm, out_hbm, slab_v, out_v):
    wid = lax.axis_index("s") * NC + lax.axis_index("c")
    pltpu.sync_copy(prod_hbm.at[:, pl.ds(wid * CPW, CPW)], slab_v)
    for g in range(BPW // L):
        col = (g * L // GT) * 128 + (g * L) % GT
        acc = jnp.zeros((L,), jnp.float32)
        for d in range(D):
            acc = acc + slab_v[d, pl.ds(col, L)]
        out_v[pl.ds(g * L, L)] = acc
    pltpu.sync_copy(out_v, out_hbm.at[pl.ds(wid * BPW, BPW)])


def kernel(inputs, word_embeddings, context_embeddings):
    idx_word = inputs[:, 1].astype(jnp.int32)
    idx_ctx = inputs[:, 0].astype(jnp.int32)
    prods = _tc_gather(idx_word, idx_ctx,
                       *([word_embeddings.T] * GT),
                       *([context_embeddings.T] * GT))
    return _sc_reduce(prods)
